# Initial kernel scaffold; baseline (speedup 1.0000x reference)
import jax
import jax.numpy as jnp
from jax.experimental import pallas as pl


def kernel(x, edge_index, W1, b1, W2, b2, W3, b3, Wc1, bc1, Wc2, bc2):
    raise NotImplementedError("write your pallas kernel here")


# speedup vs baseline: 10.5136x; 10.5136x over previous
import functools

import jax
import jax.numpy as jnp
from jax import lax
from jax.experimental import pallas as pl
from jax.experimental.pallas import tpu as pltpu
from jax.experimental.pallas import tpu_sc as plsc

N = 10000
D = 128
NE = 320000

NC = 2
NS = 16
NW = NC * NS

CH = 128
NCH = 80
E_TILE = CH * NCH
NE_PAD = E_TILE * NW
N_PAD = 10240
ROWS_PER_SUB = N_PAD // NS

_mesh = plsc.VectorSubcoreMesh(core_axis_name="c", subcore_axis_name="s")


@functools.partial(
    pl.kernel,
    mesh=_mesh,
    out_type=jax.ShapeDtypeStruct((NC, N_PAD), jnp.float32),
    scratch_types=[
        pltpu.VMEM((NCH, CH), jnp.int32),
        pltpu.VMEM((CH,), jnp.float32),
        pltpu.VMEM_SHARED((N_PAD,), jnp.float32),
    ],
)
def _sc_deg(ei3, zeros1, out, dst_idx, ones_v, acc):
    cid = lax.axis_index("c")
    sid = lax.axis_index("s")
    wid = sid * NC + cid
    for i in range(CH // 16):
        ones_v[pl.ds(i * 16, 16)] = jnp.ones((16,), jnp.float32)
    r0 = sid * ROWS_PER_SUB
    pltpu.sync_copy(zeros1.at[pl.ds(r0, ROWS_PER_SUB)],
                    acc.at[pl.ds(r0, ROWS_PER_SUB)])
    plsc.subcore_barrier()
    pltpu.sync_copy(ei3.at[1].at[pl.ds(wid * NCH, NCH)], dst_idx)

    def body(j, carry):
        pltpu.sync_copy(ones_v, acc.at[dst_idx.at[j]], add=True)
        return carry

    lax.fori_loop(0, NCH, body, 0)
    plsc.subcore_barrier()
    pltpu.sync_copy(acc.at[pl.ds(r0, ROWS_PER_SUB)],
                    out.at[cid].at[pl.ds(r0, ROWS_PER_SUB)])


@functools.partial(
    pl.kernel,
    mesh=_mesh,
    out_type=jax.ShapeDtypeStruct((NC, N_PAD), jnp.float32),
    scratch_types=[
        pltpu.VMEM((NCH, CH), jnp.int32),
        pltpu.VMEM((NCH, CH), jnp.int32),
        pltpu.VMEM((CH,), jnp.float32),
        pltpu.VMEM_SHARED((N_PAD,), jnp.float32),
        pltpu.SemaphoreType.DMA,
    ],
)
def _sc_cpre(dinv_h, ei3, zeros1, out, src_idx, dst_idx, vals, acc, sem):
    cid = lax.axis_index("c")
    sid = lax.axis_index("s")
    wid = sid * NC + cid
    r0 = sid * ROWS_PER_SUB
    pltpu.sync_copy(zeros1.at[pl.ds(r0, ROWS_PER_SUB)],
                    acc.at[pl.ds(r0, ROWS_PER_SUB)])
    plsc.subcore_barrier()
    pltpu.sync_copy(ei3.at[0].at[pl.ds(wid * NCH, NCH)], src_idx)
    pltpu.sync_copy(ei3.at[1].at[pl.ds(wid * NCH, NCH)], dst_idx)

    def body(j, carry):
        pltpu.async_copy(dinv_h.at[dst_idx.at[j]], vals, sem).wait()
        pltpu.sync_copy(vals, acc.at[src_idx.at[j]], add=True)
        return carry

    lax.fori_loop(0, NCH, body, 0)
    plsc.subcore_barrier()
    pltpu.sync_copy(acc.at[pl.ds(r0, ROWS_PER_SUB)],
                    out.at[cid].at[pl.ds(r0, ROWS_PER_SUB)])


@functools.partial(
    pl.kernel,
    mesh=_mesh,
    out_type=jax.ShapeDtypeStruct((NC, N_PAD, D), jnp.float32),
    scratch_types=[
        pltpu.VMEM((NCH, CH), jnp.int32),
        pltpu.VMEM((NCH, CH), jnp.int32),
        pltpu.VMEM((CH, D), jnp.float32),
        pltpu.VMEM_SHARED((N_PAD, D), jnp.float32),
        pltpu.SemaphoreType.DMA,
    ],
)
def _sc_agg(g_h, ei3, zeros2, out, src_idx, dst_idx, rows, acc, sem):
    cid = lax.axis_index("c")
    sid = lax.axis_index("s")
    wid = sid * NC + cid
    r0 = sid * ROWS_PER_SUB
    pltpu.sync_copy(zeros2.at[pl.ds(r0, ROWS_PER_SUB)],
                    acc.at[pl.ds(r0, ROWS_PER_SUB)])
    plsc.subcore_barrier()
    pltpu.sync_copy(ei3.at[0].at[pl.ds(wid * NCH, NCH)], src_idx)
    pltpu.sync_copy(ei3.at[1].at[pl.ds(wid * NCH, NCH)], dst_idx)

    def body(j, carry):
        pltpu.async_copy(g_h.at[src_idx.at[j]], rows, sem).wait()
        pltpu.sync_copy(rows, acc.at[dst_idx.at[j]], add=True)
        return carry

    lax.fori_loop(0, NCH, body, 0)
    plsc.subcore_barrier()
    pltpu.sync_copy(acc.at[pl.ds(r0, ROWS_PER_SUB)],
                    out.at[cid].at[pl.ds(r0, ROWS_PER_SUB)])


_BM = 640


def _tc_g_body(x_ref, w_ref, dinv_ref, o_ref):
    h = jnp.dot(x_ref[...], w_ref[...], preferred_element_type=jnp.float32)
    o_ref[...] = h * dinv_ref[...]


def _tc_g(xp, W, dinv_col):
    return pl.pallas_call(
        _tc_g_body,
        grid=(N_PAD // _BM,),
        in_specs=[
            pl.BlockSpec((_BM, D), lambda i: (i, 0)),
            pl.BlockSpec((D, D), lambda i: (0, 0)),
            pl.BlockSpec((_BM, 1), lambda i: (i, 0)),
        ],
        out_specs=pl.BlockSpec((_BM, D), lambda i: (i, 0)),
        out_shape=jax.ShapeDtypeStruct((N_PAD, D), jnp.float32),
    )(xp, W, dinv_col)


def _tc_mid_body(agg_ref, g_ref, dinv_ref, b_ref, w_ref, o_ref):
    a = agg_ref[0] + agg_ref[1] + g_ref[...]
    z = jnp.maximum(a * dinv_ref[...] + b_ref[...], 0.0)
    h = jnp.dot(z, w_ref[...], preferred_element_type=jnp.float32)
    o_ref[...] = h * dinv_ref[...]


def _tc_mid(agg, g, dinv_col, b_row, W):
    return pl.pallas_call(
        _tc_mid_body,
        grid=(N_PAD // _BM,),
        in_specs=[
            pl.BlockSpec((NC, _BM, D), lambda i: (0, i, 0)),
            pl.BlockSpec((_BM, D), lambda i: (i, 0)),
            pl.BlockSpec((_BM, 1), lambda i: (i, 0)),
            pl.BlockSpec((1, D), lambda i: (0, 0)),
            pl.BlockSpec((D, D), lambda i: (0, 0)),
        ],
        out_specs=pl.BlockSpec((_BM, D), lambda i: (i, 0)),
        out_shape=jax.ShapeDtypeStruct((N_PAD, D), jnp.float32),
    )(agg, g, dinv_col, b_row, W)


def _tc_final_body(agg_ref, g_ref, dinv_ref, b_ref, c_ref, w3_ref, b3_ref,
                   wc1_ref, bc1_ref, wc2_ref, bc2_ref, o_ref):
    a = agg_ref[0] + agg_ref[1] + g_ref[...]
    z2 = jnp.maximum(a * dinv_ref[...] + b_ref[...], 0.0)
    s = jnp.sum(z2 * c_ref[...], axis=0, keepdims=True)
    pooled = jnp.dot(s, w3_ref[...], preferred_element_type=jnp.float32)
    pooled = pooled * (1.0 / N) + b3_ref[...]
    m = jnp.maximum(
        jnp.dot(pooled, wc1_ref[...], preferred_element_type=jnp.float32)
        + bc1_ref[...], 0.0)
    o_ref[...] = (jnp.dot(m, wc2_ref[...], preferred_element_type=jnp.float32)
                  + bc2_ref[...])


def _tc_final(agg, g, dinv_col, b_row, c_col, W3, b3_row, Wc1, bc1_row,
              Wc2p, bc2p_row):
    return pl.pallas_call(
        _tc_final_body,
        out_shape=jax.ShapeDtypeStruct((1, D), jnp.float32),
    )(agg, g, dinv_col, b_row, c_col, W3, b3_row, Wc1, bc1_row, Wc2p, bc2p_row)


def kernel(x, edge_index, W1, b1, W2, b2, W3, b3, Wc1, bc1, Wc2, bc2):
    ei = edge_index.astype(jnp.int32)
    pad_cols = jnp.full((2, NE_PAD - NE), N_PAD - 1, dtype=jnp.int32)
    ei3 = jnp.concatenate([ei, pad_cols], axis=1).reshape(2, NE_PAD // CH, CH)
    xp = jnp.pad(x, ((0, N_PAD - N), (0, 0)))
    zeros1 = jnp.zeros((N_PAD,), jnp.float32)
    zeros2 = jnp.zeros((N_PAD, D), jnp.float32)

    deg_parts = _sc_deg(ei3, zeros1)
    deg = deg_parts[0] + deg_parts[1] + 1.0
    dinv = lax.rsqrt(deg)
    dinv_col = dinv[:, None]

    g1 = _tc_g(xp, W1, dinv_col)
    agg1 = _sc_agg(g1, ei3, zeros2)
    g2 = _tc_mid(agg1, g1, dinv_col, b1.reshape(1, D), W2)
    agg2 = _sc_agg(g2, ei3, zeros2)

    cparts = _sc_cpre(dinv, ei3, zeros1)
    c = dinv * (dinv + cparts[0] + cparts[1])
    c = jnp.where(jnp.arange(N_PAD) < N, c, 0.0)[:, None]

    Wc2p = jnp.pad(Wc2, ((0, 0), (0, D - Wc2.shape[1])))
    bc2p = jnp.pad(bc2, (0, D - bc2.shape[0])).reshape(1, D)
    res = _tc_final(agg2, g2, dinv_col, b2.reshape(1, D), c, W3,
                    b3.reshape(1, D), Wc1, bc1.reshape(1, Wc1.shape[1]),
                    Wc2p, bc2p)
    return res[:, :3]


# speedup vs baseline: 11.7834x; 1.1208x over previous
import functools

import jax
import jax.numpy as jnp
from jax import lax
from jax.experimental import pallas as pl
from jax.experimental.pallas import tpu as pltpu
from jax.experimental.pallas import tpu_sc as plsc

N = 10000
D = 128
NE = 320000

NC = 2
NS = 16
NW = NC * NS

CH = 128
NCH = 80
HALF = NCH // 2
E_TILE = CH * NCH
NE_PAD = E_TILE * NW
N_PAD = 10240
ROWS_PER_SUB = N_PAD // NS

_mesh = plsc.VectorSubcoreMesh(core_axis_name="c", subcore_axis_name="s")


@functools.partial(
    pl.kernel,
    mesh=_mesh,
    out_type=jax.ShapeDtypeStruct((NC, N_PAD), jnp.float32),
    scratch_types=[
        pltpu.VMEM((NCH, CH), jnp.int32),
        pltpu.VMEM((CH,), jnp.float32),
        pltpu.VMEM_SHARED((N_PAD,), jnp.float32),
    ],
)
def _sc_deg(ei3, zeros1, out, dst_idx, ones_v, acc):
    cid = lax.axis_index("c")
    sid = lax.axis_index("s")
    wid = sid * NC + cid
    for i in range(CH // 16):
        ones_v[pl.ds(i * 16, 16)] = jnp.ones((16,), jnp.float32)
    r0 = sid * ROWS_PER_SUB
    pltpu.sync_copy(zeros1.at[pl.ds(r0, ROWS_PER_SUB)],
                    acc.at[pl.ds(r0, ROWS_PER_SUB)])
    plsc.subcore_barrier()
    pltpu.sync_copy(ei3.at[1].at[pl.ds(wid * NCH, NCH)], dst_idx)

    def body(j, carry):
        pltpu.sync_copy(ones_v, acc.at[dst_idx.at[j]], add=True)
        return carry

    lax.fori_loop(0, NCH, body, 0)
    plsc.subcore_barrier()
    pltpu.sync_copy(acc.at[pl.ds(r0, ROWS_PER_SUB)],
                    out.at[cid].at[pl.ds(r0, ROWS_PER_SUB)])


@functools.partial(
    pl.kernel,
    mesh=_mesh,
    out_type=jax.ShapeDtypeStruct((NC, N_PAD), jnp.float32),
    scratch_types=[
        pltpu.VMEM((NCH, CH), jnp.int32),
        pltpu.VMEM((NCH, CH), jnp.int32),
        pltpu.VMEM((CH,), jnp.float32),
        pltpu.VMEM_SHARED((N_PAD,), jnp.float32),
        pltpu.SemaphoreType.DMA,
    ],
)
def _sc_cpre(dinv_h, ei3, zeros1, out, src_idx, dst_idx, vals, acc, sem):
    cid = lax.axis_index("c")
    sid = lax.axis_index("s")
    wid = sid * NC + cid
    r0 = sid * ROWS_PER_SUB
    pltpu.sync_copy(zeros1.at[pl.ds(r0, ROWS_PER_SUB)],
                    acc.at[pl.ds(r0, ROWS_PER_SUB)])
    plsc.subcore_barrier()
    pltpu.sync_copy(ei3.at[0].at[pl.ds(wid * NCH, NCH)], src_idx)
    pltpu.sync_copy(ei3.at[1].at[pl.ds(wid * NCH, NCH)], dst_idx)

    def body(j, carry):
        pltpu.async_copy(dinv_h.at[dst_idx.at[j]], vals, sem).wait()
        pltpu.sync_copy(vals, acc.at[src_idx.at[j]], add=True)
        return carry

    lax.fori_loop(0, NCH, body, 0)
    plsc.subcore_barrier()
    pltpu.sync_copy(acc.at[pl.ds(r0, ROWS_PER_SUB)],
                    out.at[cid].at[pl.ds(r0, ROWS_PER_SUB)])


@functools.partial(
    pl.kernel,
    mesh=_mesh,
    out_type=jax.ShapeDtypeStruct((NC, N_PAD, D), jnp.float32),
    scratch_types=[
        pltpu.VMEM((HALF, CH), jnp.int32),
        pltpu.VMEM((HALF, CH), jnp.int32),
        pltpu.VMEM((CH, D), jnp.float32),
        pltpu.VMEM((CH, D), jnp.float32),
        pltpu.VMEM_SHARED((N_PAD, D), jnp.float32),
        pltpu.SemaphoreType.DMA,
        pltpu.SemaphoreType.DMA,
    ],
)
def _sc_agg(g_h, ei3, zeros2, out, src_idx, dst_idx, rows0, rows1, acc,
            sem0, sem1):
    cid = lax.axis_index("c")
    sid = lax.axis_index("s")
    wid = sid * NC + cid
    r0 = sid * ROWS_PER_SUB
    pltpu.sync_copy(zeros2.at[pl.ds(r0, ROWS_PER_SUB)],
                    acc.at[pl.ds(r0, ROWS_PER_SUB)])
    plsc.subcore_barrier()

    for h in range(2):
        base = wid * NCH + h * HALF
        pltpu.sync_copy(ei3.at[0].at[pl.ds(base, HALF)], src_idx)
        pltpu.sync_copy(ei3.at[1].at[pl.ds(base, HALF)], dst_idx)
        pltpu.async_copy(g_h.at[src_idx.at[0]], rows0, sem0)
        pltpu.async_copy(g_h.at[src_idx.at[1]], rows1, sem1)

        def body(jj, carry):
            for b, (rows, sem) in enumerate(((rows0, sem0), (rows1, sem1))):
                j = jj * 2 + b
                pltpu.make_async_copy(g_h.at[src_idx.at[0]], rows, sem).wait()
                pltpu.sync_copy(rows, acc.at[dst_idx.at[j]], add=True)

                @pl.when(j + 2 < HALF)
                def _():
                    pltpu.async_copy(g_h.at[src_idx.at[j + 2]], rows, sem)
            return carry

        lax.fori_loop(0, HALF // 2, body, 0)
    plsc.subcore_barrier()
    pltpu.sync_copy(acc.at[pl.ds(r0, ROWS_PER_SUB)],
                    out.at[cid].at[pl.ds(r0, ROWS_PER_SUB)])


_BM = 640


def _tc_g_body(x_ref, w_ref, dinv_ref, o_ref):
    h = jnp.dot(x_ref[...], w_ref[...], preferred_element_type=jnp.float32)
    o_ref[...] = h * dinv_ref[...]


def _tc_g(xp, W, dinv_col):
    return pl.pallas_call(
        _tc_g_body,
        grid=(N_PAD // _BM,),
        in_specs=[
            pl.BlockSpec((_BM, D), lambda i: (i, 0)),
            pl.BlockSpec((D, D), lambda i: (0, 0)),
            pl.BlockSpec((_BM, 1), lambda i: (i, 0)),
        ],
        out_specs=pl.BlockSpec((_BM, D), lambda i: (i, 0)),
        out_shape=jax.ShapeDtypeStruct((N_PAD, D), jnp.float32),
    )(xp, W, dinv_col)


def _tc_mid_body(agg_ref, g_ref, dinv_ref, b_ref, w_ref, o_ref):
    a = agg_ref[0] + agg_ref[1] + g_ref[...]
    z = jnp.maximum(a * dinv_ref[...] + b_ref[...], 0.0)
    h = jnp.dot(z, w_ref[...], preferred_element_type=jnp.float32)
    o_ref[...] = h * dinv_ref[...]


def _tc_mid(agg, g, dinv_col, b_row, W):
    return pl.pallas_call(
        _tc_mid_body,
        grid=(N_PAD // _BM,),
        in_specs=[
            pl.BlockSpec((NC, _BM, D), lambda i: (0, i, 0)),
            pl.BlockSpec((_BM, D), lambda i: (i, 0)),
            pl.BlockSpec((_BM, 1), lambda i: (i, 0)),
            pl.BlockSpec((1, D), lambda i: (0, 0)),
            pl.BlockSpec((D, D), lambda i: (0, 0)),
        ],
        out_specs=pl.BlockSpec((_BM, D), lambda i: (i, 0)),
        out_shape=jax.ShapeDtypeStruct((N_PAD, D), jnp.float32),
    )(agg, g, dinv_col, b_row, W)


def _tc_final_body(agg_ref, g_ref, dinv_ref, b_ref, c_ref, w3_ref, b3_ref,
                   wc1_ref, bc1_ref, wc2_ref, bc2_ref, o_ref):
    a = agg_ref[0] + agg_ref[1] + g_ref[...]
    z2 = jnp.maximum(a * dinv_ref[...] + b_ref[...], 0.0)
    s = jnp.sum(z2 * c_ref[...], axis=0, keepdims=True)
    pooled = jnp.dot(s, w3_ref[...], preferred_element_type=jnp.float32)
    pooled = pooled * (1.0 / N) + b3_ref[...]
    m = jnp.maximum(
        jnp.dot(pooled, wc1_ref[...], preferred_element_type=jnp.float32)
        + bc1_ref[...], 0.0)
    o_ref[...] = (jnp.dot(m, wc2_ref[...], preferred_element_type=jnp.float32)
                  + bc2_ref[...])


def _tc_final(agg, g, dinv_col, b_row, c_col, W3, b3_row, Wc1, bc1_row,
              Wc2p, bc2p_row):
    return pl.pallas_call(
        _tc_final_body,
        out_shape=jax.ShapeDtypeStruct((1, D), jnp.float32),
    )(agg, g, dinv_col, b_row, c_col, W3, b3_row, Wc1, bc1_row, Wc2p, bc2p_row)


def kernel(x, edge_index, W1, b1, W2, b2, W3, b3, Wc1, bc1, Wc2, bc2):
    ei = edge_index.astype(jnp.int32)
    pad_cols = jnp.full((2, NE_PAD - NE), N_PAD - 1, dtype=jnp.int32)
    ei3 = jnp.concatenate([ei, pad_cols], axis=1).reshape(2, NE_PAD // CH, CH)
    xp = jnp.pad(x, ((0, N_PAD - N), (0, 0)))
    zeros1 = jnp.zeros((N_PAD,), jnp.float32)
    zeros2 = jnp.zeros((N_PAD, D), jnp.float32)

    deg_parts = _sc_deg(ei3, zeros1)
    deg = deg_parts[0] + deg_parts[1] + 1.0
    dinv = 1.0 / jnp.sqrt(deg)
    dinv_col = dinv[:, None]

    g1 = _tc_g(xp, W1, dinv_col)
    agg1 = _sc_agg(g1, ei3, zeros2)
    g2 = _tc_mid(agg1, g1, dinv_col, b1.reshape(1, D), W2)
    agg2 = _sc_agg(g2, ei3, zeros2)

    cparts = _sc_cpre(dinv, ei3, zeros1)
    c = dinv * (dinv + cparts[0] + cparts[1])
    c = jnp.where(jnp.arange(N_PAD) < N, c, 0.0)[:, None]

    Wc2p = jnp.pad(Wc2, ((0, 0), (0, D - Wc2.shape[1])))
    bc2p = jnp.pad(bc2, (0, D - bc2.shape[0])).reshape(1, D)
    res = _tc_final(agg2, g2, dinv_col, b2.reshape(1, D), c, W3,
                    b3.reshape(1, D), Wc1, bc1.reshape(1, Wc1.shape[1]),
                    Wc2p, bc2p)
    return res[:, :3]
